# baseline (speedup 1.0000x reference)
import functools

import jax
import jax.numpy as jnp
from jax import lax
from jax.experimental import pallas as pl
from jax.experimental.pallas import tpu as pltpu
from jax.experimental.pallas import tpu_sc as plsc

N = 50000
E = 800000
IN_FEATS = 26
D = 64
H = 4
C = 16
L = 3

NC = 2
NS = 16
AW = 40
STRIPE = 3128
N_PAD = NS * STRIPE
ET = E // NS
CH = 400
NCHUNK = ET // CH
NG = CH // 16

_SQRT_C_INV = 0.25
_B = 1000
_NB = N // _B


def _sc_edge_body(qslab, kvslab, esrc, edst, zrows, outacc,
                  acc, src_i, dst_i, qb0, qb1, kvb0, kvb1, ob0, ob1,
                  sq0, sq1, skv0, skv1, ss0, ss1):
    c = lax.axis_index("c")
    s = lax.axis_index("s")
    row0 = s * STRIPE

    pltpu.sync_copy(zrows, acc.at[pl.ds(row0, STRIPE)])
    plsc.subcore_barrier()

    coff = c * N
    qb = (qb0, qb1)
    kvb = (kvb0, kvb1)
    ob = (ob0, ob1)
    sq = (sq0, sq1)
    skv = (skv0, skv1)
    ss = (ss0, ss1)
    zero16i = jnp.zeros((16,), jnp.int32)
    zero16f = jnp.zeros((16,), jnp.float32)

    def idxvecs(g):
        sl = pl.ds(g * 16, 16)
        sv = src_i[sl] + coff
        dvr = dst_i[sl]
        return sv, dvr + coff, dvr

    def issue(g):
        p = g & 1
        sv, dv, _ = idxvecs(g)
        dq = pltpu.async_copy(qslab.at[dv], qb[p], sq[p])
        dkv = pltpu.async_copy(kvslab.at[sv], kvb[p], skv[p])
        return (dq, dkv)

    def chunk_body(k, carry):
        base = s * ET + k * CH
        pltpu.sync_copy(esrc.at[pl.ds(base, CH)], src_i)
        pltpu.sync_copy(edst.at[pl.ds(base, CH)], dst_i)
        descs = {0: issue(0)}
        sdesc = {}
        for g in range(NG):
            p = g & 1
            if g + 1 < NG:
                descs[g + 1] = issue(g + 1)
            dq, dkv = descs.pop(g)
            dq.wait()
            dkv.wait()
            if g >= 2:
                sdesc.pop(g - 2).wait()
            qref, kvref, oref = qb[p], kvb[p], ob[p]
            rows = lax.iota(jnp.int32, 16)

            def cc_body(cc, ab, qref=qref, kvref=kvref, rows=rows):
                a0, a1 = ab
                base0 = zero16i + cc * 4
                for u in range(4):
                    c0 = base0 + u
                    c1 = c0 + 16
                    a0 = a0 + (plsc.load_gather(qref, [rows, c0])
                               * plsc.load_gather(kvref, [rows, c0]))
                    a1 = a1 + (plsc.load_gather(qref, [rows, c1])
                               * plsc.load_gather(kvref, [rows, c1]))
                return (a0, a1)

            a0, a1 = lax.fori_loop(0, 4, cc_body, (zero16f, zero16f))
            e0 = jnp.exp(a0 * _SQRT_C_INV)
            e1 = jnp.exp(a1 * _SQRT_C_INV)
            plsc.store_scatter(oref, [rows, zero16i + 32], e0)
            plsc.store_scatter(oref, [rows, zero16i + 33], e1)

            for hb, ex in ((0, e0), (16, e1)):
                def col_body(col, carry2, kvref=kvref, oref=oref, rows=rows,
                             ex=ex, hb=hb):
                    colv = zero16i + (hb + col * 4)
                    for u in range(4):
                        cv = colv + u
                        v = plsc.load_gather(kvref, [rows, cv + 32])
                        plsc.store_scatter(oref, [rows, cv], v * ex)
                    return carry2

                lax.fori_loop(0, 4, col_body, 0)
            _, _, dvr = idxvecs(g)
            sdesc[g] = pltpu.async_copy(oref, acc.at[dvr], ss[p], add=True)
        sdesc.pop(NG - 2).wait()
        sdesc.pop(NG - 1).wait()
        return carry

    lax.fori_loop(0, NCHUNK, chunk_body, 0)
    plsc.subcore_barrier()
    pltpu.sync_copy(acc.at[pl.ds(row0, STRIPE)],
                    outacc.at[c, pl.ds(row0, STRIPE)])


_sc_edge = functools.partial(
    pl.kernel,
    out_type=jax.ShapeDtypeStruct((NC, N_PAD, AW), jnp.float32),
    mesh=plsc.VectorSubcoreMesh(core_axis_name="c", subcore_axis_name="s",
                                num_cores=NC, num_subcores=NS),
    compiler_params=pltpu.CompilerParams(use_tc_tiling_on_sc=False,
                                         needs_layout_passes=False),
    scratch_types=[
        pltpu.VMEM_SHARED((N_PAD, AW), jnp.float32),
        pltpu.VMEM((CH,), jnp.int32),
        pltpu.VMEM((CH,), jnp.int32),
        pltpu.VMEM((16, 32), jnp.float32),
        pltpu.VMEM((16, 32), jnp.float32),
        pltpu.VMEM((16, 64), jnp.float32),
        pltpu.VMEM((16, 64), jnp.float32),
        pltpu.VMEM((16, AW), jnp.float32),
        pltpu.VMEM((16, AW), jnp.float32),
        pltpu.SemaphoreType.DMA,
        pltpu.SemaphoreType.DMA,
        pltpu.SemaphoreType.DMA,
        pltpu.SemaphoreType.DMA,
        pltpu.SemaphoreType.DMA,
        pltpu.SemaphoreType.DMA,
    ],
)(_sc_edge_body)


def _gate_ln(h, acc, wskipT, bskip, bu, bw, g, b):
    eps = jnp.float32(1e-16)
    out = jnp.concatenate([
        acc[0, :, 0:16] / (acc[0, :, 32:33] + eps),
        acc[0, :, 16:32] / (acc[0, :, 33:34] + eps),
        acc[1, :, 0:16] / (acc[1, :, 32:33] + eps),
        acc[1, :, 16:32] / (acc[1, :, 33:34] + eps),
    ], axis=1)
    x_r = jnp.dot(h, wskipT, preferred_element_type=jnp.float32) + bskip
    logit = (jnp.dot(out, bu, preferred_element_type=jnp.float32)
             + jnp.dot(x_r, bw, preferred_element_type=jnp.float32))
    beta = jax.nn.sigmoid(logit)
    out = beta * x_r + (1.0 - beta) * out
    t = h + out
    mu = jnp.mean(t, axis=1, keepdims=True)
    var = jnp.mean((t - mu) ** 2, axis=1, keepdims=True)
    return (t - mu) * lax.rsqrt(var + 1e-5) * g + b


def _qkv_write(h_new, wqT, bq, wkT, bk, wvT, bv, qs_ref, kvs_ref):
    q = jnp.dot(h_new, wqT, preferred_element_type=jnp.float32) + bq
    k = jnp.dot(h_new, wkT, preferred_element_type=jnp.float32) + bk
    v = jnp.dot(h_new, wvT, preferred_element_type=jnp.float32) + bv
    qs_ref[0, :, :] = q[:, 0:32]
    qs_ref[1, :, :] = q[:, 32:64]
    kvs_ref[0, :, 0:32] = k[:, 0:32]
    kvs_ref[0, :, 32:64] = v[:, 0:32]
    kvs_ref[1, :, 0:32] = k[:, 32:64]
    kvs_ref[1, :, 32:64] = v[:, 32:64]


def _tc_stage0_body(x_ref, winT_ref, bin_ref, wqT_ref, bq_ref, wkT_ref,
                    bk_ref, wvT_ref, bv_ref, h_ref, qs_ref, kvs_ref):
    h = (jnp.dot(x_ref[...], winT_ref[...],
                 preferred_element_type=jnp.float32) + bin_ref[...])
    h_ref[...] = h
    _qkv_write(h, wqT_ref[...], bq_ref[...], wkT_ref[...], bk_ref[...],
               wvT_ref[...], bv_ref[...], qs_ref, kvs_ref)


def _tc_stage_mid_body(h_ref, acc_ref, wskipT_ref, bskip_ref, bu_ref, bw_ref,
                       lng_ref, lnb_ref, wqT_ref, bq_ref, wkT_ref, bk_ref,
                       wvT_ref, bv_ref, h_out_ref, qs_ref, kvs_ref):
    h_new = _gate_ln(h_ref[...], acc_ref[...], wskipT_ref[...], bskip_ref[...],
                     bu_ref[...].T, bw_ref[...].T, lng_ref[...], lnb_ref[...])
    h_out_ref[...] = h_new
    _qkv_write(h_new, wqT_ref[...], bq_ref[...], wkT_ref[...], bk_ref[...],
               wvT_ref[...], bv_ref[...], qs_ref, kvs_ref)


def _tc_stage_last_body(h_ref, acc_ref, wskipT_ref, bskip_ref, bu_ref, bw_ref,
                        lng_ref, lnb_ref, h_out_ref, gsum_ref):
    i = pl.program_id(0)
    h_new = _gate_ln(h_ref[...], acc_ref[...], wskipT_ref[...], bskip_ref[...],
                     bu_ref[...].T, bw_ref[...].T, lng_ref[...], lnb_ref[...])
    h_out_ref[...] = h_new

    @pl.when(i == 0)
    def _():
        gsum_ref[...] = jnp.zeros((8, D), jnp.float32)

    gsum_ref[0:1, :] += jnp.sum(h_new, axis=0, keepdims=True) * (1.0 / N)


def _full(shape):
    return pl.BlockSpec(shape, lambda i: tuple(0 for _ in shape))


_spec_h = pl.BlockSpec((_B, D), lambda i: (i, 0))
_spec_x = pl.BlockSpec((_B, IN_FEATS), lambda i: (i, 0))
_spec_acc = pl.BlockSpec((NC, _B, AW), lambda i: (0, i, 0))
_spec_slab = pl.BlockSpec((NC, _B, 32), lambda i: (0, i, 0))
_spec_w = _full((D, D))
_spec_b = _full((1, D))

_qkv_in_specs = [_spec_w, _spec_b, _spec_w, _spec_b, _spec_w, _spec_b]
_spec_kvslab = pl.BlockSpec((NC, _B, 64), lambda i: (0, i, 0))
_qkv_out_shapes = [jax.ShapeDtypeStruct((NC, N, 32), jnp.float32),
                   jax.ShapeDtypeStruct((NC, N, 64), jnp.float32)]
_qkv_out_specs = [_spec_slab, _spec_kvslab]

_tc_stage0 = pl.pallas_call(
    _tc_stage0_body,
    grid=(_NB,),
    in_specs=[_spec_x, _full((IN_FEATS, D)), _spec_b] + _qkv_in_specs,
    out_specs=[_spec_h] + _qkv_out_specs,
    out_shape=[jax.ShapeDtypeStruct((N, D), jnp.float32)] + _qkv_out_shapes,
)

_mid_gate_specs = [_spec_h, _spec_acc, _spec_w, _spec_b, _spec_b, _spec_b,
                   _spec_b, _spec_b]

_tc_stage_mid = pl.pallas_call(
    _tc_stage_mid_body,
    grid=(_NB,),
    in_specs=_mid_gate_specs + _qkv_in_specs,
    out_specs=[_spec_h] + _qkv_out_specs,
    out_shape=[jax.ShapeDtypeStruct((N, D), jnp.float32)] + _qkv_out_shapes,
)

_tc_stage_last = pl.pallas_call(
    _tc_stage_last_body,
    grid=(_NB,),
    in_specs=_mid_gate_specs,
    out_specs=[_spec_h, pl.BlockSpec((8, D), lambda i: (0, 0))],
    out_shape=[jax.ShapeDtypeStruct((N, D), jnp.float32),
               jax.ShapeDtypeStruct((8, D), jnp.float32)],
)


def kernel(x, edge_index, W_in, b_in, Wq, bq, Wk, bk, Wv, bv, Wskip, bskip,
           Wbeta, ln_g, ln_b):
    f32 = jnp.float32
    winT = W_in.T
    wqT = Wq.transpose(0, 2, 1)
    wkT = Wk.transpose(0, 2, 1)
    wvT = Wv.transpose(0, 2, 1)
    wskipT = Wskip.transpose(0, 2, 1)
    bu = (Wbeta[:, 0, 0:D] + Wbeta[:, 0, 2 * D:3 * D]).reshape(L, 1, D)
    bw = (Wbeta[:, 0, D:2 * D] - Wbeta[:, 0, 2 * D:3 * D]).reshape(L, 1, D)
    b2 = lambda a: a.reshape(1, D)
    zrows = jnp.zeros((STRIPE, AW), f32)

    h, qs, kvs = _tc_stage0(
        x, winT, b2(b_in), wqT[0], b2(bq[0]), wkT[0], b2(bk[0]), wvT[0],
        b2(bv[0]))

    for l in range(L):
        acc = _sc_edge(qs.reshape(NC * N, 32), kvs.reshape(NC * N, 64),
                       edge_index[0], edge_index[1], zrows)
        if l < L - 1:
            h, qs, kvs = _tc_stage_mid(
                h, acc, wskipT[l], b2(bskip[l]), bu[l], bw[l], b2(ln_g[l]),
                b2(ln_b[l]), wqT[l + 1], b2(bq[l + 1]), wkT[l + 1],
                b2(bk[l + 1]), wvT[l + 1], b2(bv[l + 1]))
        else:
            h, gsum = _tc_stage_last(
                h, acc, wskipT[l], b2(bskip[l]), bu[l], bw[l], b2(ln_g[l]),
                b2(ln_b[l]))
    g = gsum[0:1, :]
    return h, g


# speedup vs baseline: 1.0447x; 1.0447x over previous
import functools

import jax
import jax.numpy as jnp
from jax import lax
from jax.experimental import pallas as pl
from jax.experimental.pallas import tpu as pltpu
from jax.experimental.pallas import tpu_sc as plsc

N = 50000
E = 800000
IN_FEATS = 26
D = 64
H = 4
C = 16
L = 3

NC = 2
NS = 16
AW = 40
STRIPE = 3128
N_PAD = NS * STRIPE
ET = E // NS
CH = 400
NCHUNK = ET // CH
NG = CH // 16

_SQRT_C_INV = 0.25
_B = 1000
_NB = N // _B


def _sc_edge_body(qslab, kvslab, esrc, edst, zrows, outacc,
                  acc, src_i, dst_i, qb0, qb1, kvb0, kvb1, ob0, ob1,
                  sq0, sq1, skv0, skv1, ss0, ss1):
    c = lax.axis_index("c")
    s = lax.axis_index("s")
    row0 = s * STRIPE

    pltpu.sync_copy(zrows, acc.at[pl.ds(row0, STRIPE)])
    plsc.subcore_barrier()

    coff = c * N
    qb = (qb0, qb1)
    kvb = (kvb0, kvb1)
    ob = (ob0, ob1)
    sq = (sq0, sq1)
    skv = (skv0, skv1)
    ss = (ss0, ss1)
    zero16i = jnp.zeros((16,), jnp.int32)
    zero16f = jnp.zeros((16,), jnp.float32)

    def idxvecs(g):
        sl = pl.ds(g * 16, 16)
        sv = src_i[sl] + coff
        dvr = dst_i[sl]
        return sv, dvr + coff, dvr

    def issue(g):
        p = g & 1
        sv, dv, _ = idxvecs(g)
        dq = pltpu.async_copy(qslab.at[dv], qb[p], sq[p])
        dkv = pltpu.async_copy(kvslab.at[sv], kvb[p], skv[p])
        return (dq, dkv)

    def chunk_body(k, carry):
        base = s * ET + k * CH
        pltpu.sync_copy(esrc.at[pl.ds(base, CH)], src_i)
        pltpu.sync_copy(edst.at[pl.ds(base, CH)], dst_i)
        descs = {0: issue(0)}
        sdesc = {}
        for g in range(NG):
            p = g & 1
            if g + 1 < NG:
                descs[g + 1] = issue(g + 1)
            dq, dkv = descs.pop(g)
            dq.wait()
            dkv.wait()
            if g >= 2:
                sdesc.pop(g - 2).wait()
            qref, kvref, oref = qb[p], kvb[p], ob[p]
            rows = lax.iota(jnp.int32, 16)

            def cc_body(cc, ab, qref=qref, kvref=kvref, rows=rows):
                a0, a1 = ab
                c0 = zero16i + cc
                c1 = c0 + 16
                a0 = a0 + (plsc.load_gather(qref, [rows, c0])
                           * plsc.load_gather(kvref, [rows, c0]))
                a1 = a1 + (plsc.load_gather(qref, [rows, c1])
                           * plsc.load_gather(kvref, [rows, c1]))
                return (a0, a1)

            a0, a1 = lax.fori_loop(0, 16, cc_body, (zero16f, zero16f))
            e0 = jnp.exp(a0 * _SQRT_C_INV)
            e1 = jnp.exp(a1 * _SQRT_C_INV)
            plsc.store_scatter(oref, [rows, zero16i + 32], e0)
            plsc.store_scatter(oref, [rows, zero16i + 33], e1)

            def col_body(col, carry2, kvref=kvref, oref=oref, rows=rows,
                         e0=e0, e1=e1):
                colv = zero16i + col
                wt = jnp.where(col < 16, 1.0, 0.0).astype(jnp.float32)
                ex = e0 * wt + e1 * (1.0 - wt)
                v = plsc.load_gather(kvref, [rows, colv + 32])
                plsc.store_scatter(oref, [rows, colv], v * ex)
                return carry2

            lax.fori_loop(0, 32, col_body, 0)
            _, _, dvr = idxvecs(g)
            sdesc[g] = pltpu.async_copy(oref, acc.at[dvr], ss[p], add=True)
        sdesc.pop(NG - 2).wait()
        sdesc.pop(NG - 1).wait()
        return carry

    lax.fori_loop(0, NCHUNK, chunk_body, 0)
    plsc.subcore_barrier()
    pltpu.sync_copy(acc.at[pl.ds(row0, STRIPE)],
                    outacc.at[c, pl.ds(row0, STRIPE)])


_sc_edge = functools.partial(
    pl.kernel,
    out_type=jax.ShapeDtypeStruct((NC, N_PAD, AW), jnp.float32),
    mesh=plsc.VectorSubcoreMesh(core_axis_name="c", subcore_axis_name="s",
                                num_cores=NC, num_subcores=NS),
    compiler_params=pltpu.CompilerParams(use_tc_tiling_on_sc=False,
                                         needs_layout_passes=False),
    scratch_types=[
        pltpu.VMEM_SHARED((N_PAD, AW), jnp.float32),
        pltpu.VMEM((CH,), jnp.int32),
        pltpu.VMEM((CH,), jnp.int32),
        pltpu.VMEM((16, 32), jnp.float32),
        pltpu.VMEM((16, 32), jnp.float32),
        pltpu.VMEM((16, 64), jnp.float32),
        pltpu.VMEM((16, 64), jnp.float32),
        pltpu.VMEM((16, AW), jnp.float32),
        pltpu.VMEM((16, AW), jnp.float32),
        pltpu.SemaphoreType.DMA,
        pltpu.SemaphoreType.DMA,
        pltpu.SemaphoreType.DMA,
        pltpu.SemaphoreType.DMA,
        pltpu.SemaphoreType.DMA,
        pltpu.SemaphoreType.DMA,
    ],
)(_sc_edge_body)


def _gate_ln(h, acc, wskipT, bskip, bu, bw, g, b):
    eps = jnp.float32(1e-16)
    out = jnp.concatenate([
        acc[0, :, 0:16] / (acc[0, :, 32:33] + eps),
        acc[0, :, 16:32] / (acc[0, :, 33:34] + eps),
        acc[1, :, 0:16] / (acc[1, :, 32:33] + eps),
        acc[1, :, 16:32] / (acc[1, :, 33:34] + eps),
    ], axis=1)
    x_r = jnp.dot(h, wskipT, preferred_element_type=jnp.float32) + bskip
    logit = (jnp.dot(out, bu, preferred_element_type=jnp.float32)
             + jnp.dot(x_r, bw, preferred_element_type=jnp.float32))
    beta = jax.nn.sigmoid(logit)
    out = beta * x_r + (1.0 - beta) * out
    t = h + out
    mu = jnp.mean(t, axis=1, keepdims=True)
    var = jnp.mean((t - mu) ** 2, axis=1, keepdims=True)
    return (t - mu) * lax.rsqrt(var + 1e-5) * g + b


def _qkv_write(h_new, wqT, bq, wkT, bk, wvT, bv, qs_ref, kvs_ref):
    q = jnp.dot(h_new, wqT, preferred_element_type=jnp.float32) + bq
    k = jnp.dot(h_new, wkT, preferred_element_type=jnp.float32) + bk
    v = jnp.dot(h_new, wvT, preferred_element_type=jnp.float32) + bv
    qs_ref[0, :, :] = q[:, 0:32]
    qs_ref[1, :, :] = q[:, 32:64]
    kvs_ref[0, :, 0:32] = k[:, 0:32]
    kvs_ref[0, :, 32:64] = v[:, 0:32]
    kvs_ref[1, :, 0:32] = k[:, 32:64]
    kvs_ref[1, :, 32:64] = v[:, 32:64]


def _tc_stage0_body(x_ref, winT_ref, bin_ref, wqT_ref, bq_ref, wkT_ref,
                    bk_ref, wvT_ref, bv_ref, h_ref, qs_ref, kvs_ref):
    h = (jnp.dot(x_ref[...], winT_ref[...],
                 preferred_element_type=jnp.float32) + bin_ref[...])
    h_ref[...] = h
    _qkv_write(h, wqT_ref[...], bq_ref[...], wkT_ref[...], bk_ref[...],
               wvT_ref[...], bv_ref[...], qs_ref, kvs_ref)


def _tc_stage_mid_body(h_ref, acc_ref, wskipT_ref, bskip_ref, bu_ref, bw_ref,
                       lng_ref, lnb_ref, wqT_ref, bq_ref, wkT_ref, bk_ref,
                       wvT_ref, bv_ref, h_out_ref, qs_ref, kvs_ref):
    h_new = _gate_ln(h_ref[...], acc_ref[...], wskipT_ref[...], bskip_ref[...],
                     bu_ref[...].T, bw_ref[...].T, lng_ref[...], lnb_ref[...])
    h_out_ref[...] = h_new
    _qkv_write(h_new, wqT_ref[...], bq_ref[...], wkT_ref[...], bk_ref[...],
               wvT_ref[...], bv_ref[...], qs_ref, kvs_ref)


def _tc_stage_last_body(h_ref, acc_ref, wskipT_ref, bskip_ref, bu_ref, bw_ref,
                        lng_ref, lnb_ref, h_out_ref, gsum_ref):
    i = pl.program_id(0)
    h_new = _gate_ln(h_ref[...], acc_ref[...], wskipT_ref[...], bskip_ref[...],
                     bu_ref[...].T, bw_ref[...].T, lng_ref[...], lnb_ref[...])
    h_out_ref[...] = h_new

    @pl.when(i == 0)
    def _():
        gsum_ref[...] = jnp.zeros((8, D), jnp.float32)

    gsum_ref[0:1, :] += jnp.sum(h_new, axis=0, keepdims=True) * (1.0 / N)


def _full(shape):
    return pl.BlockSpec(shape, lambda i: tuple(0 for _ in shape))


_spec_h = pl.BlockSpec((_B, D), lambda i: (i, 0))
_spec_x = pl.BlockSpec((_B, IN_FEATS), lambda i: (i, 0))
_spec_acc = pl.BlockSpec((NC, _B, AW), lambda i: (0, i, 0))
_spec_slab = pl.BlockSpec((NC, _B, 32), lambda i: (0, i, 0))
_spec_w = _full((D, D))
_spec_b = _full((1, D))

_qkv_in_specs = [_spec_w, _spec_b, _spec_w, _spec_b, _spec_w, _spec_b]
_spec_kvslab = pl.BlockSpec((NC, _B, 64), lambda i: (0, i, 0))
_qkv_out_shapes = [jax.ShapeDtypeStruct((NC, N, 32), jnp.float32),
                   jax.ShapeDtypeStruct((NC, N, 64), jnp.float32)]
_qkv_out_specs = [_spec_slab, _spec_kvslab]

_tc_stage0 = pl.pallas_call(
    _tc_stage0_body,
    grid=(_NB,),
    in_specs=[_spec_x, _full((IN_FEATS, D)), _spec_b] + _qkv_in_specs,
    out_specs=[_spec_h] + _qkv_out_specs,
    out_shape=[jax.ShapeDtypeStruct((N, D), jnp.float32)] + _qkv_out_shapes,
)

_mid_gate_specs = [_spec_h, _spec_acc, _spec_w, _spec_b, _spec_b, _spec_b,
                   _spec_b, _spec_b]

_tc_stage_mid = pl.pallas_call(
    _tc_stage_mid_body,
    grid=(_NB,),
    in_specs=_mid_gate_specs + _qkv_in_specs,
    out_specs=[_spec_h] + _qkv_out_specs,
    out_shape=[jax.ShapeDtypeStruct((N, D), jnp.float32)] + _qkv_out_shapes,
)

_tc_stage_last = pl.pallas_call(
    _tc_stage_last_body,
    grid=(_NB,),
    in_specs=_mid_gate_specs,
    out_specs=[_spec_h, pl.BlockSpec((8, D), lambda i: (0, 0))],
    out_shape=[jax.ShapeDtypeStruct((N, D), jnp.float32),
               jax.ShapeDtypeStruct((8, D), jnp.float32)],
)


def kernel(x, edge_index, W_in, b_in, Wq, bq, Wk, bk, Wv, bv, Wskip, bskip,
           Wbeta, ln_g, ln_b):
    f32 = jnp.float32
    winT = W_in.T
    wqT = Wq.transpose(0, 2, 1)
    wkT = Wk.transpose(0, 2, 1)
    wvT = Wv.transpose(0, 2, 1)
    wskipT = Wskip.transpose(0, 2, 1)
    bu = (Wbeta[:, 0, 0:D] + Wbeta[:, 0, 2 * D:3 * D]).reshape(L, 1, D)
    bw = (Wbeta[:, 0, D:2 * D] - Wbeta[:, 0, 2 * D:3 * D]).reshape(L, 1, D)
    b2 = lambda a: a.reshape(1, D)
    zrows = jnp.zeros((STRIPE, AW), f32)

    h, qs, kvs = _tc_stage0(
        x, winT, b2(b_in), wqT[0], b2(bq[0]), wkT[0], b2(bk[0]), wvT[0],
        b2(bv[0]))

    for l in range(L):
        acc = _sc_edge(qs.reshape(NC * N, 32), kvs.reshape(NC * N, 64),
                       edge_index[0], edge_index[1], zrows)
        if l < L - 1:
            h, qs, kvs = _tc_stage_mid(
                h, acc, wskipT[l], b2(bskip[l]), bu[l], bw[l], b2(ln_g[l]),
                b2(ln_b[l]), wqT[l + 1], b2(bq[l + 1]), wkT[l + 1],
                b2(bk[l + 1]), wvT[l + 1], b2(bv[l + 1]))
        else:
            h, gsum = _tc_stage_last(
                h, acc, wskipT[l], b2(bskip[l]), bu[l], bw[l], b2(ln_g[l]),
                b2(ln_b[l]))
    g = gsum[0:1, :]
    return h, g
